# baseline (speedup 1.0000x reference)
import functools

import jax
import jax.numpy as jnp
from jax import lax
from jax.experimental import pallas as pl
from jax.experimental.pallas import tpu as pltpu
from jax.experimental.pallas import tpu_sc as plsc


def _build_gather(bsz, seq, d):
    info = plsc.get_sparse_core_info()
    nc, ns = info.num_cores, info.num_subcores
    nw = nc * ns
    n_rows = bsz * seq
    rows_per_w = n_rows // nw
    w_per_b = seq // rows_per_w
    chunks = [56, 56, 56, 56, 32]
    assert sum(chunks) == rows_per_w
    max_chunk = max(chunks)
    n_chunks = len(chunks)
    offs = [sum(chunks[:i]) for i in range(n_chunks)]

    mesh = plsc.VectorSubcoreMesh(core_axis_name="c", subcore_axis_name="s")

    nbuf = 2

    @functools.partial(
        pl.kernel,
        mesh=mesh,
        out_type=jax.ShapeDtypeStruct((bsz, seq, d), jnp.float32),
        scratch_types=(
            [pltpu.VMEM((rows_per_w,), jnp.int32)]
            + [pltpu.VMEM((max_chunk, d), jnp.float32) for _ in range(nbuf)]
            + [pltpu.SemaphoreType.DMA for _ in range(2 * nbuf)]
        ),
    )
    def gather_kernel(idx_hbm, table_hbm, out_hbm, idx_v, *scratch):
        bufs = scratch[:nbuf]
        gsems = scratch[nbuf:2 * nbuf]
        osems = scratch[2 * nbuf:]
        wid = lax.axis_index("s") * nc + lax.axis_index("c")
        b = wid // w_per_b
        soff = (wid % w_per_b) * rows_per_w
        pltpu.sync_copy(idx_hbm.at[b, pl.ds(soff, rows_per_w)], idx_v)

        def start_gather(c):
            buf = bufs[c % nbuf]
            return pltpu.async_copy(
                table_hbm.at[idx_v.at[pl.ds(offs[c], chunks[c])]],
                buf.at[pl.ds(0, chunks[c])], gsems[c % nbuf])

        gather = [None] * n_chunks
        out = [None] * n_chunks
        for c in range(min(nbuf, n_chunks)):
            gather[c] = start_gather(c)
        for c in range(n_chunks):
            if c >= 1 and (c - 1) + nbuf < n_chunks:
                out[c - 1].wait()
                gather[(c - 1) + nbuf] = start_gather((c - 1) + nbuf)
            gather[c].wait()
            out[c] = pltpu.async_copy(
                bufs[c % nbuf].at[pl.ds(0, chunks[c])],
                out_hbm.at[b, pl.ds(soff + offs[c], chunks[c])],
                osems[c % nbuf])
        for c in range(max(0, n_chunks - nbuf), n_chunks):
            out[c].wait()

    return gather_kernel


def kernel(token_ids, table):
    bsz, seq = token_ids.shape
    d = table.shape[1]
    gather_fn = _build_gather(bsz, seq, d)
    return gather_fn(token_ids.astype(jnp.int32), table)


# speedup vs baseline: 1.0071x; 1.0071x over previous
import functools

import jax
import jax.numpy as jnp
from jax import lax
from jax.experimental import pallas as pl
from jax.experimental.pallas import tpu as pltpu
from jax.experimental.pallas import tpu_sc as plsc


def _build_gather(bsz, seq, d):
    info = plsc.get_sparse_core_info()
    nc, ns = info.num_cores, info.num_subcores
    nw = nc * ns
    n_rows = bsz * seq
    rows_per_w = n_rows // nw
    w_per_b = seq // rows_per_w
    chunks = [8, 56, 56, 56, 56, 16, 8]
    assert sum(chunks) == rows_per_w
    max_chunk = max(chunks)
    n_chunks = len(chunks)
    offs = [sum(chunks[:i]) for i in range(n_chunks)]

    mesh = plsc.VectorSubcoreMesh(core_axis_name="c", subcore_axis_name="s")

    nbuf = 2

    @functools.partial(
        pl.kernel,
        mesh=mesh,
        out_type=jax.ShapeDtypeStruct((bsz, seq, d), jnp.float32),
        scratch_types=(
            [pltpu.VMEM((rows_per_w,), jnp.int32)]
            + [pltpu.VMEM((max_chunk, d), jnp.float32) for _ in range(nbuf)]
            + [pltpu.SemaphoreType.DMA for _ in range(2 * nbuf)]
        ),
    )
    def gather_kernel(idx_hbm, table_hbm, out_hbm, idx_v, *scratch):
        bufs = scratch[:nbuf]
        gsems = scratch[nbuf:2 * nbuf]
        osems = scratch[2 * nbuf:]
        wid = lax.axis_index("s") * nc + lax.axis_index("c")
        b = wid // w_per_b
        soff = (wid % w_per_b) * rows_per_w
        pltpu.sync_copy(idx_hbm.at[b, pl.ds(soff, rows_per_w)], idx_v)

        def start_gather(c):
            buf = bufs[c % nbuf]
            return pltpu.async_copy(
                table_hbm.at[idx_v.at[pl.ds(offs[c], chunks[c])]],
                buf.at[pl.ds(0, chunks[c])], gsems[c % nbuf])

        gather = [None] * n_chunks
        out = [None] * n_chunks
        for c in range(min(nbuf, n_chunks)):
            gather[c] = start_gather(c)
        for c in range(n_chunks):
            if c >= 1 and (c - 1) + nbuf < n_chunks:
                out[c - 1].wait()
                gather[(c - 1) + nbuf] = start_gather((c - 1) + nbuf)
            gather[c].wait()
            out[c] = pltpu.async_copy(
                bufs[c % nbuf].at[pl.ds(0, chunks[c])],
                out_hbm.at[b, pl.ds(soff + offs[c], chunks[c])],
                osems[c % nbuf])
        for c in range(max(0, n_chunks - nbuf), n_chunks):
            out[c].wait()

    return gather_kernel


def kernel(token_ids, table):
    bsz, seq = token_ids.shape
    d = table.shape[1]
    gather_fn = _build_gather(bsz, seq, d)
    return gather_fn(token_ids.astype(jnp.int32), table)


# speedup vs baseline: 1.0144x; 1.0073x over previous
import functools

import jax
import jax.numpy as jnp
from jax import lax
from jax.experimental import pallas as pl
from jax.experimental.pallas import tpu as pltpu
from jax.experimental.pallas import tpu_sc as plsc


def _build_gather(bsz, seq, d):
    info = plsc.get_sparse_core_info()
    nc, ns = info.num_cores, info.num_subcores
    nw = nc * ns
    n_rows = bsz * seq
    rows_per_w = n_rows // nw
    w_per_b = seq // rows_per_w
    chunks = [8, 56, 56, 56, 56, 24]
    assert sum(chunks) == rows_per_w
    max_chunk = max(chunks)
    n_chunks = len(chunks)
    offs = [sum(chunks[:i]) for i in range(n_chunks)]

    mesh = plsc.VectorSubcoreMesh(core_axis_name="c", subcore_axis_name="s")

    nbuf = 2

    @functools.partial(
        pl.kernel,
        mesh=mesh,
        out_type=jax.ShapeDtypeStruct((bsz, seq, d), jnp.float32),
        scratch_types=(
            [pltpu.VMEM((rows_per_w,), jnp.int32)]
            + [pltpu.VMEM((max_chunk, d), jnp.float32) for _ in range(nbuf)]
            + [pltpu.SemaphoreType.DMA for _ in range(2 * nbuf)]
        ),
    )
    def gather_kernel(idx_hbm, table_hbm, out_hbm, idx_v, *scratch):
        bufs = scratch[:nbuf]
        gsems = scratch[nbuf:2 * nbuf]
        osems = scratch[2 * nbuf:]
        wid = lax.axis_index("s") * nc + lax.axis_index("c")
        b = wid // w_per_b
        soff = (wid % w_per_b) * rows_per_w
        half = 128
        pltpu.sync_copy(idx_hbm.at[b, pl.ds(soff, half)],
                        idx_v.at[pl.ds(0, half)])

        def start_gather(c):
            buf = bufs[c % nbuf]
            return pltpu.async_copy(
                table_hbm.at[idx_v.at[pl.ds(offs[c], chunks[c])]],
                buf.at[pl.ds(0, chunks[c])], gsems[c % nbuf])

        gather = [None] * n_chunks
        out = [None] * n_chunks
        gather[0] = start_gather(0)
        pltpu.sync_copy(idx_hbm.at[b, pl.ds(soff + half, rows_per_w - half)],
                        idx_v.at[pl.ds(half, rows_per_w - half)])
        for c in range(1, min(nbuf, n_chunks)):
            gather[c] = start_gather(c)
        assert offs[min(nbuf, n_chunks) - 1] + chunks[min(nbuf, n_chunks) - 1] <= half
        for c in range(n_chunks):
            if c >= 1 and (c - 1) + nbuf < n_chunks:
                out[c - 1].wait()
                gather[(c - 1) + nbuf] = start_gather((c - 1) + nbuf)
            gather[c].wait()
            out[c] = pltpu.async_copy(
                bufs[c % nbuf].at[pl.ds(0, chunks[c])],
                out_hbm.at[b, pl.ds(soff + offs[c], chunks[c])],
                osems[c % nbuf])
        for c in range(max(0, n_chunks - nbuf), n_chunks):
            out[c].wait()

    return gather_kernel


def kernel(token_ids, table):
    bsz, seq = token_ids.shape
    d = table.shape[1]
    gather_fn = _build_gather(bsz, seq, d)
    return gather_fn(token_ids.astype(jnp.int32), table)
